# baseline (speedup 1.0000x reference)
import functools

import jax
import jax.numpy as jnp
from jax import lax
from jax.experimental import pallas as pl
from jax.experimental.pallas import tpu as pltpu
from jax.experimental.pallas import tpu_sc as plsc

H_PATCHES = 30
W_PATCHES = 40
P_OUT = 7
PADDING = 0.1
NC = 2
NS = 16
NW = NC * NS
L = 16
D = 768
N_BOX = 100
N_UNIT = N_BOX * P_OUT
UNIT_ROUNDS = (N_UNIT + NW - 1) // NW
D_CHUNKS = D // L


NBUF = 4


def _roi_body(feat_hbm, bb_hbm, iota_hbm, out_hbm,
              bb_v, iota_v, ytaps, xtaps, ywts, xwts, wv,
              idx0, idx1, idx2, idx3,
              rows0, rows1, rows2, rows3, outrow_v,
              sem0, sem1, sem2, sem3):
    cid = lax.axis_index("c")
    sid = lax.axis_index("s")
    wid = sid * NC + cid

    pltpu.sync_copy(bb_hbm, bb_v)
    pltpu.sync_copy(iota_hbm, iota_v)
    lanes = iota_v[...]
    grid = lax.iota(jnp.int32, L).astype(jnp.float32) * 0.5 + 0.25
    a_idx = lax.shift_right_logical(lanes, 2)
    c_idx = lanes & 3
    a_samp = lax.shift_right_logical(a_idx, 1)
    a_tap16 = lax.shift_left(a_idx & 1, 4)
    c_samp = lax.shift_right_logical(c_idx, 1)
    c_tap16 = lax.shift_left(c_idx & 1, 4)
    splats = [jnp.full((L,), l, jnp.int32) for l in range(2 * L)]
    bufs = (rows0, rows1, rows2, rows3)
    sems = (sem0, sem1, sem2, sem3)
    idxs = (idx0, idx1, idx2, idx3)

    box0 = ((wid >= 7).astype(jnp.int32) + (wid >= 14).astype(jnp.int32)
            + (wid >= 21).astype(jnp.int32) + (wid >= 28).astype(jnp.int32))
    p0 = wid - 7 * box0

    def round_body(r, carry):
        box, p = carry
        unit = box * P_OUT + p

        @pl.when(unit < N_UNIT)
        def _():
            b = (box >= 50).astype(jnp.int32)
            bbase = box * 16
            cx = plsc.load_gather(bb_v, [bbase + splats[4]])
            cy = plsc.load_gather(bb_v, [bbase + splats[5]])
            bw = plsc.load_gather(bb_v, [bbase + splats[6]]) \
                * (1.0 + 2.0 * PADDING)
            bh = plsc.load_gather(bb_v, [bbase + splats[7]]) \
                * (1.0 + 2.0 * PADDING)
            x1 = jnp.maximum((cx - bw * 0.5) * W_PATCHES, 0.0)
            y1 = jnp.maximum((cy - bh * 0.5) * H_PATCHES, 0.0)
            x2 = jnp.minimum((cx + bw * 0.5) * W_PATCHES, float(W_PATCHES))
            y2 = jnp.minimum((cy + bh * 0.5) * H_PATCHES, float(H_PATCHES))
            bin_w = (x2 - x1) * (1.0 / P_OUT)
            bin_h = (y2 - y1) * (1.0 / P_OUT)
            xs = (x1 - 0.5) + bin_w * grid
            ys = (y1 - 0.5) + bin_h * grid
            xc = jnp.clip(xs, 0.0, float(W_PATCHES - 1))
            yc = jnp.clip(ys, 0.0, float(H_PATCHES - 1))
            x0 = xc.astype(jnp.int32)
            y0 = yc.astype(jnp.int32)
            lx = xc - x0.astype(jnp.float32)
            ly = yc - y0.astype(jnp.float32)
            xtaps[pl.ds(0, L)] = x0
            xtaps[pl.ds(L, L)] = jnp.minimum(x0 + 1, W_PATCHES - 1)
            xwts[pl.ds(0, L)] = 1.0 - lx
            xwts[pl.ds(L, L)] = lx
            ytaps[pl.ds(0, L)] = y0
            ytaps[pl.ds(L, L)] = jnp.minimum(y0 + 1, H_PATCHES - 1)
            ywts[pl.ds(0, L)] = 1.0 - ly
            ywts[pl.ds(L, L)] = ly

            iy = (2 * p + a_samp) + a_tap16
            yt = plsc.load_gather(ytaps, [iy])
            wy = plsc.load_gather(ywts, [iy])
            rowbase = b * (H_PATCHES * W_PATCHES) + yt * W_PATCHES

            def bin_idx(q):
                ix = (2 * q + c_samp) + c_tap16
                xt = plsc.load_gather(xtaps, [ix])
                wx = plsc.load_gather(xwts, [ix])
                return rowbase + xt, wy * wx * 0.25

            def issue(q):
                ridx, w = bin_idx(q)
                s = q % NBUF
                idxs[s][...] = ridx
                return pltpu.async_copy(feat_hbm.at[idxs[s]],
                                        bufs[s], sems[s]), w

            pend = []
            wlist = []
            for q in range(NBUF - 1):
                d, w = issue(q)
                pend.append(d)
                wlist.append(w)
            for q in range(P_OUT):
                if q + NBUF - 1 < P_OUT:
                    dn, wn = issue(q + NBUF - 1)
                    pend.append(dn)
                    wlist.append(wn)
                wv[pl.ds(L, L)] = wlist[q]
                ws = [plsc.load_gather(wv, [splats[L + l]]) for l in range(L)]
                pend[q].wait()
                buf = bufs[q % NBUF]

                def c_loop(c, _):
                    for u in range(2):
                        sl = pl.ds((2 * c + u) * L, L)
                        acc = ws[0] * buf[0, sl]
                        for l in range(1, L):
                            acc = acc + ws[l] * buf[l, sl]
                        outrow_v[pl.ds(q * D + (2 * c + u) * L, L)] = acc
                    return 0

                lax.fori_loop(0, D_CHUNKS // 2, c_loop, 0)
            pltpu.sync_copy(outrow_v, out_hbm.at[unit])

        p_n = p + 4
        over = (p_n >= P_OUT).astype(jnp.int32)
        return (box + 4 + over, p_n - P_OUT * over)

    lax.fori_loop(0, UNIT_ROUNDS, round_body, (box0, p0))


@jax.jit
def kernel(spatial_features, bboxes):
    B, HW, Dd = spatial_features.shape
    _, N, _ = bboxes.shape
    feat = spatial_features.reshape(B * HW, Dd)
    bb_pad = jnp.zeros((N_BOX, 16), jnp.float32).at[:, 4:8].set(
        bboxes.reshape(N_BOX, 4))

    mesh = plsc.VectorSubcoreMesh(core_axis_name="c", subcore_axis_name="s",
                                  num_cores=NC, num_subcores=NS)
    out = pl.kernel(
        _roi_body,
        out_type=jax.ShapeDtypeStruct((N_UNIT, P_OUT * D), jnp.float32),
        mesh=mesh,
        compiler_params=pltpu.CompilerParams(needs_layout_passes=False),
        scratch_types=[
            pltpu.VMEM((N_BOX * 16,), jnp.float32),
            pltpu.VMEM((L,), jnp.int32),
            pltpu.VMEM((2 * L,), jnp.int32),
            pltpu.VMEM((2 * L,), jnp.int32),
            pltpu.VMEM((2 * L,), jnp.float32),
            pltpu.VMEM((2 * L,), jnp.float32),
            pltpu.VMEM((2 * L,), jnp.float32),
            pltpu.VMEM((L,), jnp.int32),
            pltpu.VMEM((L,), jnp.int32),
            pltpu.VMEM((L,), jnp.int32),
            pltpu.VMEM((L,), jnp.int32),
            pltpu.VMEM((L, D), jnp.float32),
            pltpu.VMEM((L, D), jnp.float32),
            pltpu.VMEM((L, D), jnp.float32),
            pltpu.VMEM((L, D), jnp.float32),
            pltpu.VMEM((P_OUT * D,), jnp.float32),
            pltpu.SemaphoreType.DMA,
            pltpu.SemaphoreType.DMA,
            pltpu.SemaphoreType.DMA,
            pltpu.SemaphoreType.DMA,
        ],
    )(feat, bb_pad.reshape(N_BOX * 16), jnp.arange(L, dtype=jnp.int32))
    return out.reshape(B, N, P_OUT * P_OUT, Dd)


# speedup vs baseline: 1.0306x; 1.0306x over previous
import functools

import jax
import jax.numpy as jnp
from jax import lax
from jax.experimental import pallas as pl
from jax.experimental.pallas import tpu as pltpu
from jax.experimental.pallas import tpu_sc as plsc

H_PATCHES = 30
W_PATCHES = 40
P_OUT = 7
PADDING = 0.1
NC = 2
NS = 16
NW = NC * NS
L = 16
D = 768
N_BOX = 100
N_UNIT = N_BOX * P_OUT
UNIT_ROUNDS = (N_UNIT + NW - 1) // NW
D_CHUNKS = D // L


NBUF = 2


def _roi_body(feat_hbm, bb_hbm, iota_hbm, out_hbm,
              bb_v, iota_v, ytaps, xtaps, ywts, xwts, wv,
              idx0, idx1, idx2, idx3,
              rows0, rows1, rows2, rows3, outrow_v,
              sem0, sem1, sem2, sem3):
    cid = lax.axis_index("c")
    sid = lax.axis_index("s")
    wid = sid * NC + cid

    pltpu.sync_copy(bb_hbm, bb_v)
    pltpu.sync_copy(iota_hbm, iota_v)
    lanes = iota_v[...]
    grid = lax.iota(jnp.int32, L).astype(jnp.float32) * 0.5 + 0.25
    a_idx = lax.shift_right_logical(lanes, 2)
    c_idx = lanes & 3
    a_samp = lax.shift_right_logical(a_idx, 1)
    a_tap16 = lax.shift_left(a_idx & 1, 4)
    c_samp = lax.shift_right_logical(c_idx, 1)
    c_tap16 = lax.shift_left(c_idx & 1, 4)
    splats = [jnp.full((L,), l, jnp.int32) for l in range(2 * L)]
    bufs = (rows0, rows1, rows2, rows3)
    sems = (sem0, sem1, sem2, sem3)
    idxs = (idx0, idx1, idx2, idx3)

    box0 = ((wid >= 7).astype(jnp.int32) + (wid >= 14).astype(jnp.int32)
            + (wid >= 21).astype(jnp.int32) + (wid >= 28).astype(jnp.int32))
    p0 = wid - 7 * box0

    def round_body(r, carry):
        box, p = carry
        unit = box * P_OUT + p

        @pl.when(unit < N_UNIT)
        def _():
            b = (box >= 50).astype(jnp.int32)
            bbase = box * 16
            cx = plsc.load_gather(bb_v, [bbase + splats[4]])
            cy = plsc.load_gather(bb_v, [bbase + splats[5]])
            bw = plsc.load_gather(bb_v, [bbase + splats[6]]) \
                * (1.0 + 2.0 * PADDING)
            bh = plsc.load_gather(bb_v, [bbase + splats[7]]) \
                * (1.0 + 2.0 * PADDING)
            x1 = jnp.maximum((cx - bw * 0.5) * W_PATCHES, 0.0)
            y1 = jnp.maximum((cy - bh * 0.5) * H_PATCHES, 0.0)
            x2 = jnp.minimum((cx + bw * 0.5) * W_PATCHES, float(W_PATCHES))
            y2 = jnp.minimum((cy + bh * 0.5) * H_PATCHES, float(H_PATCHES))
            bin_w = (x2 - x1) * (1.0 / P_OUT)
            bin_h = (y2 - y1) * (1.0 / P_OUT)
            xs = (x1 - 0.5) + bin_w * grid
            ys = (y1 - 0.5) + bin_h * grid
            xc = jnp.clip(xs, 0.0, float(W_PATCHES - 1))
            yc = jnp.clip(ys, 0.0, float(H_PATCHES - 1))
            x0 = xc.astype(jnp.int32)
            y0 = yc.astype(jnp.int32)
            lx = xc - x0.astype(jnp.float32)
            ly = yc - y0.astype(jnp.float32)
            xtaps[pl.ds(0, L)] = x0
            xtaps[pl.ds(L, L)] = jnp.minimum(x0 + 1, W_PATCHES - 1)
            xwts[pl.ds(0, L)] = 1.0 - lx
            xwts[pl.ds(L, L)] = lx
            ytaps[pl.ds(0, L)] = y0
            ytaps[pl.ds(L, L)] = jnp.minimum(y0 + 1, H_PATCHES - 1)
            ywts[pl.ds(0, L)] = 1.0 - ly
            ywts[pl.ds(L, L)] = ly

            iy = (2 * p + a_samp) + a_tap16
            yt = plsc.load_gather(ytaps, [iy])
            wy = plsc.load_gather(ywts, [iy])
            rowbase = b * (H_PATCHES * W_PATCHES) + yt * W_PATCHES

            def bin_idx(q):
                ix = (2 * q + c_samp) + c_tap16
                xt = plsc.load_gather(xtaps, [ix])
                wx = plsc.load_gather(xwts, [ix])
                return rowbase + xt, wy * wx * 0.25

            def issue(q):
                ridx, w = bin_idx(q)
                s = q % NBUF
                idxs[s][...] = ridx
                return pltpu.async_copy(feat_hbm.at[idxs[s]],
                                        bufs[s], sems[s]), w

            pend = []
            wlist = []
            for q in range(NBUF - 1):
                d, w = issue(q)
                pend.append(d)
                wlist.append(w)
            for q in range(P_OUT):
                if q + NBUF - 1 < P_OUT:
                    dn, wn = issue(q + NBUF - 1)
                    pend.append(dn)
                    wlist.append(wn)
                wv[pl.ds(L, L)] = wlist[q]
                ws = [plsc.load_gather(wv, [splats[L + l]]) for l in range(L)]
                pend[q].wait()
                buf = bufs[q % NBUF]

                def c_loop(c, _):
                    for u in range(2):
                        sl = pl.ds((2 * c + u) * L, L)
                        acc = ws[0] * buf[0, sl]
                        for l in range(1, L):
                            acc = acc + ws[l] * buf[l, sl]
                        outrow_v[pl.ds(q * D + (2 * c + u) * L, L)] = acc
                    return 0

                lax.fori_loop(0, D_CHUNKS // 2, c_loop, 0)
            pltpu.sync_copy(outrow_v, out_hbm.at[unit])

        p_n = p + 4
        over = (p_n >= P_OUT).astype(jnp.int32)
        return (box + 4 + over, p_n - P_OUT * over)

    lax.fori_loop(0, UNIT_ROUNDS, round_body, (box0, p0))


@jax.jit
def kernel(spatial_features, bboxes):
    B, HW, Dd = spatial_features.shape
    _, N, _ = bboxes.shape
    feat = spatial_features.reshape(B * HW, Dd)
    bb_pad = jnp.zeros((N_BOX, 16), jnp.float32).at[:, 4:8].set(
        bboxes.reshape(N_BOX, 4))

    mesh = plsc.VectorSubcoreMesh(core_axis_name="c", subcore_axis_name="s",
                                  num_cores=NC, num_subcores=NS)
    out = pl.kernel(
        _roi_body,
        out_type=jax.ShapeDtypeStruct((N_UNIT, P_OUT * D), jnp.float32),
        mesh=mesh,
        compiler_params=pltpu.CompilerParams(needs_layout_passes=False),
        scratch_types=[
            pltpu.VMEM((N_BOX * 16,), jnp.float32),
            pltpu.VMEM((L,), jnp.int32),
            pltpu.VMEM((2 * L,), jnp.int32),
            pltpu.VMEM((2 * L,), jnp.int32),
            pltpu.VMEM((2 * L,), jnp.float32),
            pltpu.VMEM((2 * L,), jnp.float32),
            pltpu.VMEM((2 * L,), jnp.float32),
            pltpu.VMEM((L,), jnp.int32),
            pltpu.VMEM((L,), jnp.int32),
            pltpu.VMEM((L,), jnp.int32),
            pltpu.VMEM((L,), jnp.int32),
            pltpu.VMEM((L, D), jnp.float32),
            pltpu.VMEM((L, D), jnp.float32),
            pltpu.VMEM((L, D), jnp.float32),
            pltpu.VMEM((L, D), jnp.float32),
            pltpu.VMEM((P_OUT * D,), jnp.float32),
            pltpu.SemaphoreType.DMA,
            pltpu.SemaphoreType.DMA,
            pltpu.SemaphoreType.DMA,
            pltpu.SemaphoreType.DMA,
        ],
    )(feat, bb_pad.reshape(N_BOX * 16), jnp.arange(L, dtype=jnp.int32))
    return out.reshape(B, N, P_OUT * P_OUT, Dd)
